# Initial kernel scaffold; baseline (speedup 1.0000x reference)
import jax
import jax.numpy as jnp
from jax.experimental import pallas as pl


def kernel(L, relative_bias):
    raise NotImplementedError("write your pallas kernel here")


# speedup vs baseline: 42.9913x; 42.9913x over previous
import jax
import jax.numpy as jnp
import numpy as np
from jax import lax
from jax.experimental import pallas as pl
from jax.experimental.pallas import tpu as pltpu
from jax.experimental.pallas import tpu_sc as plsc

_H = 16
_L = 2048
_TW = 4096
_PITCH = 4104
_NSH = 8
_NW = 32
_ROWS_PER_W = (_H * _L) // _NW
_U = 8


def _sc_body(tab_hbm, out_hbm, col_v, tab_v, sem):
    c = lax.axis_index("c")
    s = lax.axis_index("s")
    wid = s * 2 + c
    h = wid // 2
    i0 = (wid % 2) * _ROWS_PER_W

    pltpu.sync_copy(tab_hbm.at[h], col_v)
    pltpu.sync_copy(tab_hbm.at[h], tab_v.at[pl.ds(0, _TW)])

    lanes = lax.iota(jnp.int32, 16)

    def build(k, carry):
        base = k * 16
        start_a = pl.multiple_of(jnp.maximum(base - 16, 0), 16)
        a = col_v[pl.ds(start_a, 16)]
        b = col_v[pl.ds(pl.multiple_of(base, 16), 16)]
        for sh in range(1, _NSH):
            idx_a = (16 - sh + lanes) % 16
            idx_b = (lanes - sh) % 16
            pa = jnp.take(a, idx_a, mode="wrap")
            pb = jnp.take(b, idx_b, mode="wrap")
            w = jnp.where(lanes < sh, pa, pb)
            tab_v[pl.ds(pl.multiple_of(sh * _PITCH + base, 8), 16)] = w
        return carry

    lax.fori_loop(0, _TW // 16, build, 0)

    def chunk(g, carry):
        cps = []
        for u in range(_U):
            i = i0 + g * _U + u
            sh = (i + 1) % _NSH
            a = sh * _PITCH + (_L - 1) - i + sh
            cps.append(
                pltpu.async_copy(
                    tab_v.at[pl.ds(pl.multiple_of(a, 8), _L)],
                    out_hbm.at[pl.ds(pl.multiple_of((h * _L + i) * _L, 8), _L)],
                    sem,
                )
            )
        for cp in cps:
            cp.wait()
        return carry

    lax.fori_loop(0, _ROWS_PER_W // _U, chunk, 0)


def kernel(L, relative_bias):
    del L
    tab_t = jnp.zeros((_H, _TW), jnp.float32).at[:, : 2 * _L - 1].set(relative_bias.T)
    mesh = plsc.VectorSubcoreMesh(core_axis_name="c", subcore_axis_name="s")
    run = pl.kernel(
        _sc_body,
        out_type=jax.ShapeDtypeStruct((_H * _L * _L,), jnp.float32),
        mesh=mesh,
        scratch_types=[
            pltpu.VMEM((_TW,), jnp.float32),
            pltpu.VMEM((_NSH * _PITCH,), jnp.float32),
            pltpu.SemaphoreType.DMA,
        ],
    )
    return run(tab_t).reshape(_H, _L, _L)


# speedup vs baseline: 43.4609x; 1.0109x over previous
import jax
import jax.numpy as jnp
import numpy as np
from jax import lax
from jax.experimental import pallas as pl
from jax.experimental.pallas import tpu as pltpu
from jax.experimental.pallas import tpu_sc as plsc

_H = 16
_L = 2048
_TW = 4096
_PITCH = 4112
_NSH = 16
_NW = 32
_ROWS_PER_W = (_H * _L) // _NW
_U = 16


def _sc_body(tab_hbm, out_hbm, col_v, tab_v, sem):
    c = lax.axis_index("c")
    s = lax.axis_index("s")
    wid = s * 2 + c
    h = wid // 2
    i0 = (wid % 2) * _ROWS_PER_W

    pltpu.sync_copy(tab_hbm.at[h], col_v)
    pltpu.sync_copy(tab_hbm.at[h], tab_v.at[pl.ds(0, _TW)])

    lanes = lax.iota(jnp.int32, 16)

    def build(k, carry):
        base = k * 16
        start_a = pl.multiple_of(jnp.maximum(base - 16, 0), 16)
        a = col_v[pl.ds(start_a, 16)]
        b = col_v[pl.ds(pl.multiple_of(base, 16), 16)]
        for sh in range(1, _NSH):
            idx_a = (16 - sh + lanes) % 16
            idx_b = (lanes - sh) % 16
            pa = jnp.take(a, idx_a, mode="wrap")
            pb = jnp.take(b, idx_b, mode="wrap")
            w = jnp.where(lanes < sh, pa, pb)
            tab_v[pl.ds(pl.multiple_of(sh * _PITCH + base, 16), 16)] = w
        return carry

    lax.fori_loop(0, _TW // 16, build, 0)

    def chunk(g, carry):
        cps = []
        for u in range(_U):
            i = i0 + g * _U + u
            sh = (i + 1) % _NSH
            a = sh * _PITCH + (_L - 1) - i + sh
            cps.append(
                pltpu.async_copy(
                    tab_v.at[pl.ds(pl.multiple_of(a, 16), _L)],
                    out_hbm.at[pl.ds(pl.multiple_of((h * _L + i) * _L, 8), _L)],
                    sem,
                )
            )
        for cp in cps:
            cp.wait()
        return carry

    lax.fori_loop(0, _ROWS_PER_W // _U, chunk, 0)


def kernel(L, relative_bias):
    del L
    tab_t = jnp.zeros((_H, _TW), jnp.float32).at[:, : 2 * _L - 1].set(relative_bias.T)
    mesh = plsc.VectorSubcoreMesh(core_axis_name="c", subcore_axis_name="s")
    run = pl.kernel(
        _sc_body,
        out_type=jax.ShapeDtypeStruct((_H * _L * _L,), jnp.float32),
        mesh=mesh,
        scratch_types=[
            pltpu.VMEM((_TW,), jnp.float32),
            pltpu.VMEM((_NSH * _PITCH,), jnp.float32),
            pltpu.SemaphoreType.DMA,
        ],
    )
    return run(tab_t).reshape(_H, _L, _L)


# speedup vs baseline: 80.6506x; 1.8557x over previous
import jax
import jax.numpy as jnp
from jax import lax
from jax.experimental import pallas as pl
from jax.experimental.pallas import tpu as pltpu
from jax.experimental.pallas import tpu_sc as plsc

_H = 16
_L = 2048
_TW = 4096
_SLOT = 4096
_NSL = 8
_HPC = 8
_BLK = _L // 128


def _sc_body(tab_hbm, out_hbm, col_v, tab_v, sem):
    c = lax.axis_index("c")
    t = lax.axis_index("s")
    lanes = lax.iota(jnp.int32, 16)

    def build(hh_next, p):
        h_next = jnp.minimum(c * _HPC + hh_next, _H - 1)
        pltpu.sync_copy(tab_hbm.at[h_next], col_v)
        base_p = p * (_NSL * _SLOT)

        def bchunk(k, cc):
            x = 128 + k * 16
            for u in range(_NSL):
                sh = 8 * t + 1 + u
                q = sh // 16
                r = sh % 16
                off_b = pl.multiple_of(x - q * 16, 16)
                off_a = pl.multiple_of(jnp.maximum(x - q * 16 - 16, 0), 16)
                va = col_v[pl.ds(off_a, 16)]
                vb = col_v[pl.ds(off_b, 16)]
                pa = jnp.take(va, (16 - r + lanes) % 16, mode="wrap")
                pb = jnp.take(vb, (lanes - r) % 16, mode="wrap")
                w = jnp.where(lanes < r, pa, pb)
                tab_v[pl.ds(pl.multiple_of(base_p + u * _SLOT + x, 16), 16)] = w
            return cc

        lax.fori_loop(0, (_TW - 128) // 16, bchunk, 0)

    build(0, 0)

    def head_iter(hh, carry):
        p = hh % 2
        h = c * _HPC + hh
        cps = []
        for b in range(_BLK):
            for u in range(_NSL):
                i = 128 * b + 8 * t + u
                src = pl.multiple_of(
                    p * (_NSL * _SLOT) + u * _SLOT + (_L - 128 * b), 128
                )
                cps.append(
                    pltpu.async_copy(
                        tab_v.at[pl.ds(src, _L)], out_hbm.at[h, i], sem
                    )
                )

        @pl.when(hh + 1 < _HPC)
        def _():
            build(hh + 1, 1 - p)

        for cp in cps:
            cp.wait()
        return carry

    lax.fori_loop(0, _HPC, head_iter, 0)


def kernel(L, relative_bias):
    del L
    tab_t = jnp.zeros((_H, _TW), jnp.float32).at[:, : 2 * _L - 1].set(relative_bias.T)
    mesh = plsc.VectorSubcoreMesh(core_axis_name="c", subcore_axis_name="s")
    run = pl.kernel(
        _sc_body,
        out_type=jax.ShapeDtypeStruct((_H, _L, _L), jnp.float32),
        mesh=mesh,
        scratch_types=[
            pltpu.VMEM((_TW,), jnp.float32),
            pltpu.VMEM((2 * _NSL * _SLOT,), jnp.float32),
            pltpu.SemaphoreType.DMA,
        ],
    )
    return run(tab_t)


# speedup vs baseline: 107.2952x; 1.3304x over previous
import jax
import jax.numpy as jnp
from jax import lax
from jax.experimental import pallas as pl
from jax.experimental.pallas import tpu as pltpu
from jax.experimental.pallas import tpu_sc as plsc

_H = 16
_L = 2048
_TW = 4096
_SLOT = 4096
_NSL = 8
_HPC = 8
_BLK = _L // 128


def _sc_body(tab_hbm, out_hbm, col_v, tab_v, sem):
    c = lax.axis_index("c")
    t = lax.axis_index("s")
    lanes = lax.iota(jnp.int32, 16)

    def bchunk_group(p, k_lo, k_hi):
        base_p = p * (_NSL * _SLOT)

        def bchunk(k, cc):
            x = 128 + k * 16
            for u in range(_NSL):
                sh = 8 * t + 1 + u
                q = sh // 16
                r = sh % 16
                off_b = pl.multiple_of(x - q * 16, 16)
                off_a = pl.multiple_of(jnp.maximum(x - q * 16 - 16, 0), 16)
                va = col_v[pl.ds(off_a, 16)]
                vb = col_v[pl.ds(off_b, 16)]
                pa = jnp.take(va, (16 - r + lanes) % 16, mode="wrap")
                pb = jnp.take(vb, (lanes - r) % 16, mode="wrap")
                w = jnp.where(lanes < r, pa, pb)
                tab_v[pl.ds(pl.multiple_of(base_p + u * _SLOT + x, 16), 16)] = w
            return cc

        lax.fori_loop(k_lo, k_hi, bchunk, 0)

    _NCH = (_TW - 128) // 16
    _GRP = -(-_NCH // _BLK)

    pltpu.sync_copy(tab_hbm.at[c * _HPC], col_v)
    bchunk_group(0, 0, _NCH)

    def head_iter(hh, carry):
        p = hh % 2
        h = c * _HPC + hh

        @pl.when(hh + 1 < _HPC)
        def _():
            pltpu.sync_copy(tab_hbm.at[c * _HPC + hh + 1], col_v)

        cps = []
        for b in range(_BLK):
            for u in range(_NSL):
                i = 128 * b + 8 * t + u
                src = pl.multiple_of(
                    p * (_NSL * _SLOT) + u * _SLOT + (_L - 128 * b), 128
                )
                cps.append(
                    pltpu.async_copy(
                        tab_v.at[pl.ds(src, _L)], out_hbm.at[h, i], sem
                    )
                )
            k_lo, k_hi = b * _GRP, min((b + 1) * _GRP, _NCH)
            if k_lo < k_hi:

                @pl.when(hh + 1 < _HPC)
                def _(k_lo=k_lo, k_hi=k_hi):
                    bchunk_group(1 - p, k_lo, k_hi)

        for cp in cps:
            cp.wait()
        return carry

    lax.fori_loop(0, _HPC, head_iter, 0)


def kernel(L, relative_bias):
    del L
    tab_t = jnp.zeros((_H, _TW), jnp.float32).at[:, : 2 * _L - 1].set(relative_bias.T)
    mesh = plsc.VectorSubcoreMesh(core_axis_name="c", subcore_axis_name="s")
    run = pl.kernel(
        _sc_body,
        out_type=jax.ShapeDtypeStruct((_H, _L, _L), jnp.float32),
        mesh=mesh,
        scratch_types=[
            pltpu.VMEM((_TW,), jnp.float32),
            pltpu.VMEM((2 * _NSL * _SLOT,), jnp.float32),
            pltpu.SemaphoreType.DMA,
        ],
    )
    return run(tab_t)


# speedup vs baseline: 107.4850x; 1.0018x over previous
import jax
import jax.numpy as jnp
from jax import lax
from jax.experimental import pallas as pl
from jax.experimental.pallas import tpu as pltpu
from jax.experimental.pallas import tpu_sc as plsc

_H = 16
_L = 2048
_TW = 4096
_SLOT = 4096
_NSL = 8
_HPC = 8
_BLK = _L // 128


def _sc_body(tab_hbm, out_hbm, col_v, tab_v, sem):
    c = lax.axis_index("c")
    t = lax.axis_index("s")
    lanes = lax.iota(jnp.int32, 16)

    qoff, ia, ib, msk = [], [], [], []
    for u in range(_NSL):
        sh = 8 * t + 1 + u
        r = sh % 16
        qoff.append((sh // 16) * 16)
        ia.append((16 - r + lanes) % 16)
        ib.append((lanes - r) % 16)
        msk.append(lanes < r)

    def bchunk_group(p, k_lo, k_hi):
        base_p = p * (_NSL * _SLOT)

        def bchunk(k, cc):
            x = 128 + k * 16
            for u in range(_NSL):
                off_b = pl.multiple_of(x - qoff[u], 16)
                off_a = pl.multiple_of(jnp.maximum(x - qoff[u] - 16, 0), 16)
                va = col_v[pl.ds(off_a, 16)]
                vb = col_v[pl.ds(off_b, 16)]
                pa = jnp.take(va, ia[u], mode="wrap")
                pb = jnp.take(vb, ib[u], mode="wrap")
                w = jnp.where(msk[u], pa, pb)
                tab_v[pl.ds(pl.multiple_of(base_p + u * _SLOT + x, 16), 16)] = w
            return cc

        lax.fori_loop(k_lo, k_hi, bchunk, 0)

    _NCH = (_TW - 128) // 16
    _GRP = -(-_NCH // _BLK)

    pltpu.sync_copy(tab_hbm.at[c * _HPC], col_v)
    bchunk_group(0, 0, _NCH)

    def head_iter(hh, carry):
        p = hh % 2
        h = c * _HPC + hh

        @pl.when(hh + 1 < _HPC)
        def _():
            pltpu.sync_copy(tab_hbm.at[c * _HPC + hh + 1], col_v)

        cps = []
        for b in range(_BLK):
            for u in range(_NSL):
                i = 128 * b + 8 * t + u
                src = pl.multiple_of(
                    p * (_NSL * _SLOT) + u * _SLOT + (_L - 128 * b), 128
                )
                cps.append(
                    pltpu.async_copy(
                        tab_v.at[pl.ds(src, _L)], out_hbm.at[h, i], sem
                    )
                )
            k_lo, k_hi = b * _GRP, min((b + 1) * _GRP, _NCH)
            if k_lo < k_hi:

                @pl.when(hh + 1 < _HPC)
                def _(k_lo=k_lo, k_hi=k_hi):
                    bchunk_group(1 - p, k_lo, k_hi)

        for cp in cps:
            cp.wait()
        return carry

    lax.fori_loop(0, _HPC, head_iter, 0)


def kernel(L, relative_bias):
    del L
    tab_t = jnp.zeros((_H, _TW), jnp.float32).at[:, : 2 * _L - 1].set(relative_bias.T)
    mesh = plsc.VectorSubcoreMesh(core_axis_name="c", subcore_axis_name="s")
    run = pl.kernel(
        _sc_body,
        out_type=jax.ShapeDtypeStruct((_H, _L, _L), jnp.float32),
        mesh=mesh,
        scratch_types=[
            pltpu.VMEM((_TW,), jnp.float32),
            pltpu.VMEM((2 * _NSL * _SLOT,), jnp.float32),
            pltpu.SemaphoreType.DMA,
        ],
    )
    return run(tab_t)
